# Initial kernel scaffold; baseline (speedup 1.0000x reference)
import jax
import jax.numpy as jnp
from jax.experimental import pallas as pl


def kernel(x, Ws1, Ws2, We1, We2, Wg):
    raise NotImplementedError("write your pallas kernel here")


# speedup vs baseline: 1.0332x; 1.0332x over previous
import functools

import jax
import jax.numpy as jnp
from jax import lax
from jax.experimental import pallas as pl
from jax.experimental.pallas import tpu as pltpu
from jax.experimental.pallas import tpu_sc as plsc

N = 2048
D = 768
E = 8
H = 3072
TM = 128
NTOK_TILES = N // TM
NTILES_MOE = N * 2 // TM + 8
CAP = NTILES_MOE * TM
NC = 2
NS = 16
NW = NC * NS
TPW = N // NW
CHUNK = 32


def _routing_body(x_ref, wg_ref, e0_ref, e1_ref, g0_ref, g1_ref,
                  r0_ref, r1_ref, poff_ref, te_ref, carry):
    i = pl.program_id(0)

    @pl.when(i == 0)
    def _():
        carry[...] = jnp.zeros_like(carry)

    logits = jnp.dot(x_ref[...], wg_ref[...],
                     preferred_element_type=jnp.float32)
    iota_e = lax.broadcasted_iota(jnp.int32, (TM, E), 1)
    m0 = jnp.max(logits, axis=1, keepdims=True)
    e0 = jnp.min(jnp.where(logits == m0, iota_e, E), axis=1, keepdims=True)
    oh0 = (iota_e == e0).astype(jnp.float32)
    masked = jnp.where(iota_e == e0, -jnp.inf, logits)
    m1 = jnp.max(masked, axis=1, keepdims=True)
    e1 = jnp.min(jnp.where(masked == m1, iota_e, E), axis=1, keepdims=True)
    oh1 = (iota_e == e1).astype(jnp.float32)
    g0 = 1.0 / (1.0 + jnp.exp(m1 - m0))

    tot = oh0 + oh1
    ir = lax.broadcasted_iota(jnp.int32, (TM, TM), 0)
    ic = lax.broadcasted_iota(jnp.int32, (TM, TM), 1)
    tri = (ic < ir).astype(jnp.float32)
    cum = carry[...] + jnp.dot(tri, tot, preferred_element_type=jnp.float32)
    r0 = jnp.sum(cum * oh0, axis=1, keepdims=True)
    r1 = jnp.sum(cum * oh1, axis=1, keepdims=True)

    e0_ref[...] = e0
    e1_ref[...] = e1
    g0_ref[...] = g0
    g1_ref[...] = 1.0 - g0
    r0_ref[...] = r0
    r1_ref[...] = r1

    newcarry = carry[...] + jnp.sum(tot, axis=0, keepdims=True)
    carry[...] = newcarry

    @pl.when(i == NTOK_TILES - 1)
    def _():
        cnt = newcarry
        ptiles = jnp.floor((cnt + (TM - 1)) / TM)
        ia = lax.broadcasted_iota(jnp.int32, (E, E), 0)
        ib = lax.broadcasted_iota(jnp.int32, (E, E), 1)
        strict = (ia < ib).astype(jnp.float32)
        pcum = jnp.dot(ptiles, strict, preferred_element_type=jnp.float32)
        poff_ref[...] = pcum * TM
        pincl = pcum + ptiles
        tio = lax.broadcasted_iota(jnp.int32, (1, 64), 1).astype(jnp.float32)
        te = jnp.zeros((1, 64), jnp.float32)
        for e in range(E):
            te = te + (tio >= pincl[:, e:e + 1]).astype(jnp.float32)
        te_ref[...] = jnp.minimum(te, 7.0).astype(jnp.int32)


def _routing(xf, Wg):
    out_shapes = (
        jax.ShapeDtypeStruct((N, 1), jnp.int32),
        jax.ShapeDtypeStruct((N, 1), jnp.int32),
        jax.ShapeDtypeStruct((N, 1), jnp.float32),
        jax.ShapeDtypeStruct((N, 1), jnp.float32),
        jax.ShapeDtypeStruct((N, 1), jnp.float32),
        jax.ShapeDtypeStruct((N, 1), jnp.float32),
        jax.ShapeDtypeStruct((1, E), jnp.float32),
        jax.ShapeDtypeStruct((1, 64), jnp.int32),
    )
    tile_spec = pl.BlockSpec((TM, 1), lambda i: (i, 0))
    return pl.pallas_call(
        _routing_body,
        grid=(NTOK_TILES,),
        in_specs=[pl.BlockSpec((TM, D), lambda i: (i, 0)),
                  pl.BlockSpec((D, E), lambda i: (0, 0))],
        out_specs=(tile_spec, tile_spec, tile_spec, tile_spec,
                   tile_spec, tile_spec,
                   pl.BlockSpec((1, E), lambda i: (0, 0)),
                   pl.BlockSpec((1, 64), lambda i: (0, 0))),
        out_shape=out_shapes,
        scratch_shapes=[pltpu.VMEM((1, E), jnp.float32)],
    )(xf, Wg)


def _dispatch_body(x_hbm, e0_hbm, e1_hbm, r0_hbm, r1_hbm, poff_hbm,
                   xs_hbm, pos0_hbm, pos1_hbm,
                   poff_v, e_v, r_v, idx0_v, idx1_v, rows_v, sem):
    wid = lax.axis_index("s") * NC + lax.axis_index("c")
    base = wid * TPW
    pltpu.sync_copy(poff_hbm, poff_v)

    pltpu.sync_copy(e0_hbm.at[pl.ds(base, TPW)], e_v)
    pltpu.sync_copy(r0_hbm.at[pl.ds(base, TPW)], r_v)
    for j in range(TPW // 16):
        ev = e_v[pl.ds(16 * j, 16)]
        idx0_v[pl.ds(16 * j, 16)] = (plsc.load_gather(poff_v, [ev])
                                     + r_v[pl.ds(16 * j, 16)])
    pltpu.sync_copy(e1_hbm.at[pl.ds(base, TPW)], e_v)
    pltpu.sync_copy(r1_hbm.at[pl.ds(base, TPW)], r_v)
    for j in range(TPW // 16):
        ev = e_v[pl.ds(16 * j, 16)]
        idx1_v[pl.ds(16 * j, 16)] = (plsc.load_gather(poff_v, [ev])
                                     + r_v[pl.ds(16 * j, 16)])

    pltpu.sync_copy(x_hbm.at[pl.ds(base, TPW)], rows_v)
    pltpu.async_copy(rows_v, xs_hbm.at[idx0_v], sem).wait()
    pltpu.async_copy(rows_v, xs_hbm.at[idx1_v], sem).wait()
    pltpu.sync_copy(idx0_v, pos0_hbm.at[pl.ds(base, TPW)])
    pltpu.sync_copy(idx1_v, pos1_hbm.at[pl.ds(base, TPW)])


def _dispatch(xf, e0i, e1i, r0i, r1i, poffi):
    mesh = plsc.VectorSubcoreMesh(core_axis_name="c", subcore_axis_name="s")
    f = functools.partial(
        pl.kernel,
        out_type=(jax.ShapeDtypeStruct((CAP, D), jnp.float32),
                  jax.ShapeDtypeStruct((N,), jnp.int32),
                  jax.ShapeDtypeStruct((N,), jnp.int32)),
        mesh=mesh,
        scratch_types=[pltpu.VMEM((16,), jnp.int32),
                       pltpu.VMEM((TPW,), jnp.int32),
                       pltpu.VMEM((TPW,), jnp.int32),
                       pltpu.VMEM((TPW,), jnp.int32),
                       pltpu.VMEM((TPW,), jnp.int32),
                       pltpu.VMEM((TPW, D), jnp.float32),
                       pltpu.SemaphoreType.DMA],
        compiler_params=pltpu.CompilerParams(needs_layout_passes=False),
    )(_dispatch_body)
    return f(xf, e0i, e1i, r0i, r1i, poffi)


def _gmm_body(te_ref, xs_ref, w1_ref, w2_ref, ys_ref):
    h = jax.nn.silu(jnp.dot(xs_ref[...], w1_ref[0],
                            preferred_element_type=jnp.float32))
    ys_ref[...] = jnp.dot(h, w2_ref[0], preferred_element_type=jnp.float32)


def _gmm(te, xs, We1, We2):
    grid_spec = pltpu.PrefetchScalarGridSpec(
        num_scalar_prefetch=1,
        grid=(NTILES_MOE,),
        in_specs=[pl.BlockSpec((TM, D), lambda t, te_ref: (t, 0)),
                  pl.BlockSpec((1, D, H), lambda t, te_ref: (te_ref[t], 0, 0)),
                  pl.BlockSpec((1, H, D), lambda t, te_ref: (te_ref[t], 0, 0))],
        out_specs=pl.BlockSpec((TM, D), lambda t, te_ref: (t, 0)),
    )
    return pl.pallas_call(
        _gmm_body,
        grid_spec=grid_spec,
        out_shape=jax.ShapeDtypeStruct((CAP, D), jnp.float32),
    )(te, xs, We1, We2)


def _shared_body(x_ref, w1_ref, w2_ref, o_ref):
    h = jax.nn.silu(jnp.dot(x_ref[...], w1_ref[...],
                            preferred_element_type=jnp.float32))
    o_ref[...] = jnp.dot(h, w2_ref[...], preferred_element_type=jnp.float32)


def _shared(xf, Ws1, Ws2):
    return pl.pallas_call(
        _shared_body,
        grid=(NTOK_TILES,),
        in_specs=[pl.BlockSpec((TM, D), lambda i: (i, 0)),
                  pl.BlockSpec((D, H), lambda i: (0, 0)),
                  pl.BlockSpec((H, D), lambda i: (0, 0))],
        out_specs=pl.BlockSpec((TM, D), lambda i: (i, 0)),
        out_shape=jax.ShapeDtypeStruct((N, D), jnp.float32),
    )(xf, Ws1, Ws2)


def _combine_body(ys_hbm, sh_hbm, pos0_hbm, pos1_hbm, g0_hbm, g1_hbm,
                  out_hbm,
                  idx0_v, idx1_v, g0_v, g1_v, y0_v, y1_v, o_v, sem):
    wid = lax.axis_index("s") * NC + lax.axis_index("c")
    for c in range(TPW // CHUNK):
        base = wid * TPW + c * CHUNK
        pltpu.sync_copy(pos0_hbm.at[pl.ds(base, CHUNK)], idx0_v)
        pltpu.sync_copy(pos1_hbm.at[pl.ds(base, CHUNK)], idx1_v)
        pltpu.sync_copy(g0_hbm.at[pl.ds(base, CHUNK)], g0_v)
        pltpu.sync_copy(g1_hbm.at[pl.ds(base, CHUNK)], g1_v)
        pltpu.async_copy(ys_hbm.at[idx0_v], y0_v, sem).wait()
        pltpu.async_copy(ys_hbm.at[idx1_v], y1_v, sem).wait()
        pltpu.sync_copy(sh_hbm.at[pl.ds(base, CHUNK)], o_v)

        def token_body(j, _):
            gj0 = plsc.load_gather(g0_v, [jnp.full((16,), j, jnp.int32)])
            gj1 = plsc.load_gather(g1_v, [jnp.full((16,), j, jnp.int32)])
            for k in range(D // 16):
                sl = pl.ds(16 * k, 16)
                o_v[j, sl] = (o_v[j, sl] + gj0 * y0_v[j, sl]
                              + gj1 * y1_v[j, sl])
            return 0

        lax.fori_loop(0, CHUNK, token_body, 0)
        pltpu.sync_copy(o_v, out_hbm.at[pl.ds(base, CHUNK)])


def _combine(ys, sh, pos0, pos1, g0, g1):
    mesh = plsc.VectorSubcoreMesh(core_axis_name="c", subcore_axis_name="s")
    f = functools.partial(
        pl.kernel,
        out_type=jax.ShapeDtypeStruct((N, D), jnp.float32),
        mesh=mesh,
        scratch_types=[pltpu.VMEM((CHUNK,), jnp.int32),
                       pltpu.VMEM((CHUNK,), jnp.int32),
                       pltpu.VMEM((CHUNK,), jnp.float32),
                       pltpu.VMEM((CHUNK,), jnp.float32),
                       pltpu.VMEM((CHUNK, D), jnp.float32),
                       pltpu.VMEM((CHUNK, D), jnp.float32),
                       pltpu.VMEM((CHUNK, D), jnp.float32),
                       pltpu.SemaphoreType.DMA],
        compiler_params=pltpu.CompilerParams(needs_layout_passes=False),
    )(_combine_body)
    return f(ys, sh, pos0, pos1, g0, g1)


def kernel(x, Ws1, Ws2, We1, We2, Wg):
    Bn, Sn, Dn = x.shape
    xf = x.reshape(N, D)
    e0, e1, g0, g1, r0, r1, poff, te = _routing(xf, Wg)
    e0i = e0.reshape(N)
    e1i = e1.reshape(N)
    r0i = r0.reshape(N).astype(jnp.int32)
    r1i = r1.reshape(N).astype(jnp.int32)
    poffi = jnp.pad(poff.reshape(E).astype(jnp.int32), (0, 16 - E))
    te_arr = te.reshape(64)

    xs, pos0, pos1 = _dispatch(xf, e0i, e1i, r0i, r1i, poffi)
    ys = _gmm(te_arr, xs, We1, We2)
    sh = _shared(xf, Ws1, Ws2)
    out = _combine(ys, sh, pos0, pos1, g0.reshape(N), g1.reshape(N))
    return out.reshape(Bn, Sn, Dn)


# speedup vs baseline: 1.0728x; 1.0383x over previous
import functools

import jax
import jax.numpy as jnp
from jax import lax
from jax.experimental import pallas as pl
from jax.experimental.pallas import tpu as pltpu
from jax.experimental.pallas import tpu_sc as plsc

N = 2048
D = 768
E = 8
H = 3072
TM = 128
NTOK_TILES = N // TM
NTILES_MOE = N * 2 // TM + 8
CAP = NTILES_MOE * TM
NTILES_ALL = NTILES_MOE + NTOK_TILES
XROWS = CAP + N
NC = 2
NS = 16
NW = NC * NS
TPW = N // NW
CHUNK = 32


def _routing_body(x_ref, wg_ref, e0_ref, e1_ref, g0_ref, g1_ref,
                  r0_ref, r1_ref, poff_ref, te_ref, first_ref, par_ref,
                  nxt_ref, carry):
    i = pl.program_id(0)

    @pl.when(i == 0)
    def _():
        carry[...] = jnp.zeros_like(carry)

    logits = jnp.dot(x_ref[...], wg_ref[...],
                     preferred_element_type=jnp.float32)
    iota_e = lax.broadcasted_iota(jnp.int32, (TM, E), 1)
    m0 = jnp.max(logits, axis=1, keepdims=True)
    e0 = jnp.min(jnp.where(logits == m0, iota_e, E), axis=1, keepdims=True)
    oh0 = (iota_e == e0).astype(jnp.float32)
    masked = jnp.where(iota_e == e0, -jnp.inf, logits)
    m1 = jnp.max(masked, axis=1, keepdims=True)
    e1 = jnp.min(jnp.where(masked == m1, iota_e, E), axis=1, keepdims=True)
    oh1 = (iota_e == e1).astype(jnp.float32)
    g0 = 1.0 / (1.0 + jnp.exp(m1 - m0))

    tot = oh0 + oh1
    ir = lax.broadcasted_iota(jnp.int32, (TM, TM), 0)
    ic = lax.broadcasted_iota(jnp.int32, (TM, TM), 1)
    tri = (ic < ir).astype(jnp.float32)
    cum = carry[...] + jnp.dot(tri, tot, preferred_element_type=jnp.float32)
    r0 = jnp.sum(cum * oh0, axis=1, keepdims=True)
    r1 = jnp.sum(cum * oh1, axis=1, keepdims=True)

    e0_ref[...] = e0
    e1_ref[...] = e1
    g0_ref[...] = g0
    g1_ref[...] = 1.0 - g0
    r0_ref[...] = r0
    r1_ref[...] = r1

    newcarry = carry[...] + jnp.sum(tot, axis=0, keepdims=True)
    carry[...] = newcarry

    @pl.when(i == NTOK_TILES - 1)
    def _():
        cnt = newcarry
        ptiles = jnp.floor((cnt + (TM - 1)) / TM)
        ia = lax.broadcasted_iota(jnp.int32, (E, E), 0)
        ib = lax.broadcasted_iota(jnp.int32, (E, E), 1)
        strict = (ia < ib).astype(jnp.float32)
        pcum = jnp.dot(ptiles, strict, preferred_element_type=jnp.float32)
        poff_ref[...] = pcum * TM
        pincl = pcum + ptiles
        tio = lax.broadcasted_iota(jnp.int32, (1, 64), 1).astype(jnp.float32)
        te = jnp.zeros((1, 64), jnp.float32)
        for e in range(E):
            te = te + (tio >= pincl[:, e:e + 1]).astype(jnp.float32)
        te_ref[...] = te.astype(jnp.int32)

        ia64 = lax.broadcasted_iota(jnp.int32, (64, 64), 0)
        ib64 = lax.broadcasted_iota(jnp.int32, (64, 64), 1)
        shift = (ia64 == ib64 - 1).astype(jnp.float32)
        te_prev = jnp.dot(te, shift, preferred_element_type=jnp.float32)
        first = jnp.where((tio == 0.0) | (te_prev != te), 1.0, 0.0)
        incl = (ia64 <= ib64).astype(jnp.float32)
        runidx = jnp.dot(first, incl, preferred_element_type=jnp.float32)
        rm1 = runidx - 1.0
        par = rm1 - 2.0 * jnp.floor(rm1 * 0.5)
        nxt = jnp.full((1, 64), float(E + 1), jnp.float32)
        for e in range(E, -1, -1):
            if e == E:
                pres = jnp.full((1, 1), 1.0, jnp.float32)
            else:
                pres = (ptiles[:, e:e + 1] > 0.0).astype(jnp.float32)
            take = (te < float(e)) & (pres > 0.0)
            nxt = jnp.where(take, float(e), nxt)
        first_ref[...] = first.astype(jnp.int32)
        par_ref[...] = par.astype(jnp.int32)
        nxt_ref[...] = nxt.astype(jnp.int32)


def _routing(xf, Wg):
    out_shapes = (
        jax.ShapeDtypeStruct((N, 1), jnp.int32),
        jax.ShapeDtypeStruct((N, 1), jnp.int32),
        jax.ShapeDtypeStruct((N, 1), jnp.float32),
        jax.ShapeDtypeStruct((N, 1), jnp.float32),
        jax.ShapeDtypeStruct((N, 1), jnp.float32),
        jax.ShapeDtypeStruct((N, 1), jnp.float32),
        jax.ShapeDtypeStruct((1, E), jnp.float32),
        jax.ShapeDtypeStruct((1, 64), jnp.int32),
        jax.ShapeDtypeStruct((1, 64), jnp.int32),
        jax.ShapeDtypeStruct((1, 64), jnp.int32),
        jax.ShapeDtypeStruct((1, 64), jnp.int32),
    )
    tile_spec = pl.BlockSpec((TM, 1), lambda i: (i, 0))
    small_spec = pl.BlockSpec((1, 64), lambda i: (0, 0))
    return pl.pallas_call(
        _routing_body,
        grid=(NTOK_TILES,),
        in_specs=[pl.BlockSpec((TM, D), lambda i: (i, 0)),
                  pl.BlockSpec((D, E), lambda i: (0, 0))],
        out_specs=(tile_spec, tile_spec, tile_spec, tile_spec,
                   tile_spec, tile_spec,
                   pl.BlockSpec((1, E), lambda i: (0, 0)),
                   small_spec, small_spec, small_spec, small_spec),
        out_shape=out_shapes,
        scratch_shapes=[pltpu.VMEM((1, E), jnp.float32)],
    )(xf, Wg)


def _dispatch_body(x_hbm, e0_hbm, e1_hbm, r0_hbm, r1_hbm, poff_hbm,
                   xs_hbm, pos0_hbm, pos1_hbm,
                   poff_v, e_v, r_v, idx0_v, idx1_v, rows_v, sem):
    wid = lax.axis_index("s") * NC + lax.axis_index("c")
    base = wid * TPW
    pltpu.sync_copy(poff_hbm, poff_v)

    pltpu.sync_copy(e0_hbm.at[pl.ds(base, TPW)], e_v)
    pltpu.sync_copy(r0_hbm.at[pl.ds(base, TPW)], r_v)
    for j in range(TPW // 16):
        ev = e_v[pl.ds(16 * j, 16)]
        idx0_v[pl.ds(16 * j, 16)] = (plsc.load_gather(poff_v, [ev])
                                     + r_v[pl.ds(16 * j, 16)])
    pltpu.sync_copy(e1_hbm.at[pl.ds(base, TPW)], e_v)
    pltpu.sync_copy(r1_hbm.at[pl.ds(base, TPW)], r_v)
    for j in range(TPW // 16):
        ev = e_v[pl.ds(16 * j, 16)]
        idx1_v[pl.ds(16 * j, 16)] = (plsc.load_gather(poff_v, [ev])
                                     + r_v[pl.ds(16 * j, 16)])

    pltpu.sync_copy(x_hbm.at[pl.ds(base, TPW)], rows_v)
    pltpu.async_copy(rows_v, xs_hbm.at[idx0_v], sem).wait()
    pltpu.async_copy(rows_v, xs_hbm.at[idx1_v], sem).wait()
    pltpu.sync_copy(rows_v, xs_hbm.at[pl.ds(CAP + base, TPW)])
    pltpu.sync_copy(idx0_v, pos0_hbm.at[pl.ds(base, TPW)])
    pltpu.sync_copy(idx1_v, pos1_hbm.at[pl.ds(base, TPW)])


def _dispatch(xf, e0i, e1i, r0i, r1i, poffi):
    mesh = plsc.VectorSubcoreMesh(core_axis_name="c", subcore_axis_name="s")
    f = functools.partial(
        pl.kernel,
        out_type=(jax.ShapeDtypeStruct((XROWS, D), jnp.float32),
                  jax.ShapeDtypeStruct((N,), jnp.int32),
                  jax.ShapeDtypeStruct((N,), jnp.int32)),
        mesh=mesh,
        scratch_types=[pltpu.VMEM((16,), jnp.int32),
                       pltpu.VMEM((TPW,), jnp.int32),
                       pltpu.VMEM((TPW,), jnp.int32),
                       pltpu.VMEM((TPW,), jnp.int32),
                       pltpu.VMEM((TPW,), jnp.int32),
                       pltpu.VMEM((TPW, D), jnp.float32),
                       pltpu.SemaphoreType.DMA],
        compiler_params=pltpu.CompilerParams(needs_layout_passes=False),
    )(_dispatch_body)
    return f(xf, e0i, e1i, r0i, r1i, poffi)


def _issue_load(e, b, we1, we2, ws1, ws2, w1buf, w2buf, sem1, sem2):
    @pl.when(e < E)
    def _():
        pltpu.make_async_copy(we1.at[e], w1buf.at[b], sem1.at[b]).start()
        pltpu.make_async_copy(we2.at[e], w2buf.at[b], sem2.at[b]).start()

    @pl.when(e == E)
    def _():
        pltpu.make_async_copy(ws1, w1buf.at[b], sem1.at[b]).start()
        pltpu.make_async_copy(ws2, w2buf.at[b], sem2.at[b]).start()


def _gmm_body(te_s, first_s, par_s, nxt_s, xs_ref, we1, we2, ws1, ws2,
              ys_ref, w1buf, w2buf, sem1, sem2):
    t = pl.program_id(0)

    @pl.when(t == 0)
    def _():
        _issue_load(te_s[0], 0, we1, we2, ws1, ws2, w1buf, w2buf, sem1, sem2)

    @pl.when(first_s[t] == 1)
    def _():
        b = par_s[t]
        pltpu.make_async_copy(we1.at[0], w1buf.at[b], sem1.at[b]).wait()
        pltpu.make_async_copy(we2.at[0], w2buf.at[b], sem2.at[b]).wait()
        _issue_load(nxt_s[t], 1 - b, we1, we2, ws1, ws2, w1buf, w2buf,
                    sem1, sem2)

    b = par_s[t]
    h = jax.nn.silu(jnp.dot(xs_ref[...], w1buf[b],
                            preferred_element_type=jnp.float32))
    ys_ref[...] = jnp.dot(h, w2buf[b], preferred_element_type=jnp.float32)


def _gmm(te, first, par, nxt, xs, We1, We2, Ws1, Ws2):
    grid_spec = pltpu.PrefetchScalarGridSpec(
        num_scalar_prefetch=4,
        grid=(NTILES_ALL,),
        in_specs=[pl.BlockSpec((TM, D), lambda t, *_: (t, 0)),
                  pl.BlockSpec(memory_space=pl.ANY),
                  pl.BlockSpec(memory_space=pl.ANY),
                  pl.BlockSpec(memory_space=pl.ANY),
                  pl.BlockSpec(memory_space=pl.ANY)],
        out_specs=pl.BlockSpec((TM, D), lambda t, *_: (t, 0)),
        scratch_shapes=[pltpu.VMEM((2, D, H), jnp.float32),
                        pltpu.VMEM((2, H, D), jnp.float32),
                        pltpu.SemaphoreType.DMA((2,)),
                        pltpu.SemaphoreType.DMA((2,))],
    )
    return pl.pallas_call(
        _gmm_body,
        grid_spec=grid_spec,
        out_shape=jax.ShapeDtypeStruct((XROWS, D), jnp.float32),
        compiler_params=pltpu.CompilerParams(
            vmem_limit_bytes=100 * 1024 * 1024),
    )(te, first, par, nxt, xs, We1, We2, Ws1, Ws2)


def _combine_body(ys_hbm, pos0_hbm, pos1_hbm, g0_hbm, g1_hbm,
                  out_hbm,
                  idx0_v, idx1_v, g0_v, g1_v, y0_v, y1_v, o_v, sem):
    wid = lax.axis_index("s") * NC + lax.axis_index("c")
    for c in range(TPW // CHUNK):
        base = wid * TPW + c * CHUNK
        pltpu.sync_copy(pos0_hbm.at[pl.ds(base, CHUNK)], idx0_v)
        pltpu.sync_copy(pos1_hbm.at[pl.ds(base, CHUNK)], idx1_v)
        pltpu.sync_copy(g0_hbm.at[pl.ds(base, CHUNK)], g0_v)
        pltpu.sync_copy(g1_hbm.at[pl.ds(base, CHUNK)], g1_v)
        pltpu.async_copy(ys_hbm.at[idx0_v], y0_v, sem).wait()
        pltpu.async_copy(ys_hbm.at[idx1_v], y1_v, sem).wait()
        pltpu.sync_copy(ys_hbm.at[pl.ds(CAP + base, CHUNK)], o_v)

        def token_body(j, _):
            gj0 = plsc.load_gather(g0_v, [jnp.full((16,), j, jnp.int32)])
            gj1 = plsc.load_gather(g1_v, [jnp.full((16,), j, jnp.int32)])
            for k in range(D // 16):
                sl = pl.ds(16 * k, 16)
                o_v[j, sl] = (o_v[j, sl] + gj0 * y0_v[j, sl]
                              + gj1 * y1_v[j, sl])
            return 0

        lax.fori_loop(0, CHUNK, token_body, 0)
        pltpu.sync_copy(o_v, out_hbm.at[pl.ds(base, CHUNK)])


def _combine(ys, pos0, pos1, g0, g1):
    mesh = plsc.VectorSubcoreMesh(core_axis_name="c", subcore_axis_name="s")
    f = functools.partial(
        pl.kernel,
        out_type=jax.ShapeDtypeStruct((N, D), jnp.float32),
        mesh=mesh,
        scratch_types=[pltpu.VMEM((CHUNK,), jnp.int32),
                       pltpu.VMEM((CHUNK,), jnp.int32),
                       pltpu.VMEM((CHUNK,), jnp.float32),
                       pltpu.VMEM((CHUNK,), jnp.float32),
                       pltpu.VMEM((CHUNK, D), jnp.float32),
                       pltpu.VMEM((CHUNK, D), jnp.float32),
                       pltpu.VMEM((CHUNK, D), jnp.float32),
                       pltpu.SemaphoreType.DMA],
        compiler_params=pltpu.CompilerParams(needs_layout_passes=False),
    )(_combine_body)
    return f(ys, pos0, pos1, g0, g1)


def kernel(x, Ws1, Ws2, We1, We2, Wg):
    Bn, Sn, Dn = x.shape
    xf = x.reshape(N, D)
    e0, e1, g0, g1, r0, r1, poff, te, first, par, nxt = _routing(xf, Wg)
    e0i = e0.reshape(N)
    e1i = e1.reshape(N)
    r0i = r0.reshape(N).astype(jnp.int32)
    r1i = r1.reshape(N).astype(jnp.int32)
    poffi = jnp.pad(poff.reshape(E).astype(jnp.int32), (0, 16 - E))

    xs, pos0, pos1 = _dispatch(xf, e0i, e1i, r0i, r1i, poffi)
    ys = _gmm(te.reshape(64), first.reshape(64), par.reshape(64),
              nxt.reshape(64), xs, We1, We2, Ws1, Ws2)
    out = _combine(ys, pos0, pos1, g0.reshape(N), g1.reshape(N))
    return out.reshape(Bn, Sn, Dn)


# speedup vs baseline: 1.2016x; 1.1201x over previous
import functools

import jax
import jax.numpy as jnp
from jax import lax
from jax.experimental import pallas as pl
from jax.experimental.pallas import tpu as pltpu
from jax.experimental.pallas import tpu_sc as plsc

N = 2048
D = 768
E = 8
H = 3072
TM = 128
NTOK_TILES = N // TM
NTILES_MOE = N * 2 // TM + 8
CAP = NTILES_MOE * TM
NTILES_ALL = NTILES_MOE + NTOK_TILES
XROWS = CAP + N
NC = 2
NS = 16
NW = NC * NS
TPW = N // NW
CHUNK = 16
NCHUNK = TPW // CHUNK
TMR = 512
NTILES_R = N // TMR


def _routing_body(x_ref, wg_ref, pk_ref, poff_ref, te_ref, first_ref,
                  par_ref, nxt_ref, carry):
    i = pl.program_id(0)

    @pl.when(i == 0)
    def _():
        carry[...] = jnp.zeros_like(carry)

    logits = jnp.dot(x_ref[...], wg_ref[...],
                     preferred_element_type=jnp.float32)
    iota_e = lax.broadcasted_iota(jnp.int32, (TMR, E), 1)
    m0 = jnp.max(logits, axis=1, keepdims=True)
    e0 = jnp.min(jnp.where(logits == m0, iota_e, E), axis=1, keepdims=True)
    oh0 = (iota_e == e0).astype(jnp.float32)
    masked = jnp.where(iota_e == e0, -jnp.inf, logits)
    m1 = jnp.max(masked, axis=1, keepdims=True)
    e1 = jnp.min(jnp.where(masked == m1, iota_e, E), axis=1, keepdims=True)
    oh1 = (iota_e == e1).astype(jnp.float32)
    g0 = 1.0 / (1.0 + jnp.exp(m1 - m0))

    tot = oh0 + oh1
    ir = lax.broadcasted_iota(jnp.int32, (TMR, TMR), 0)
    ic = lax.broadcasted_iota(jnp.int32, (TMR, TMR), 1)
    tri = (ic < ir).astype(jnp.float32)
    cum = carry[...] + jnp.dot(tri, tot, preferred_element_type=jnp.float32)
    r0 = jnp.sum(cum * oh0, axis=1, keepdims=True)
    r1 = jnp.sum(cum * oh1, axis=1, keepdims=True)

    lane = lax.broadcasted_iota(jnp.int32, (TMR, E), 1)
    pk = jnp.where(lane == 0, e0.astype(jnp.float32), 0.0)
    pk = jnp.where(lane == 1, e1.astype(jnp.float32), pk)
    pk = jnp.where(lane == 2, g0, pk)
    pk = jnp.where(lane == 3, 1.0 - g0, pk)
    pk = jnp.where(lane == 4, r0, pk)
    pk_ref[...] = jnp.where(lane == 5, r1, pk)

    newcarry = carry[...] + jnp.sum(tot, axis=0, keepdims=True)
    carry[...] = newcarry

    @pl.when(i == NTILES_R - 1)
    def _():
        cnt = newcarry
        ptiles = jnp.floor((cnt + (TM - 1)) / TM)
        ia = lax.broadcasted_iota(jnp.int32, (E, E), 0)
        ib = lax.broadcasted_iota(jnp.int32, (E, E), 1)
        strict = (ia < ib).astype(jnp.float32)
        pcum = jnp.dot(ptiles, strict, preferred_element_type=jnp.float32)
        poff_ref[...] = pcum * TM
        pincl = pcum + ptiles
        tio = lax.broadcasted_iota(jnp.int32, (1, 64), 1).astype(jnp.float32)
        te = jnp.zeros((1, 64), jnp.float32)
        for e in range(E):
            te = te + (tio >= pincl[:, e:e + 1]).astype(jnp.float32)
        te_ref[...] = te.astype(jnp.int32)

        ia64 = lax.broadcasted_iota(jnp.int32, (64, 64), 0)
        ib64 = lax.broadcasted_iota(jnp.int32, (64, 64), 1)
        shift = (ia64 == ib64 - 1).astype(jnp.float32)
        te_prev = jnp.dot(te, shift, preferred_element_type=jnp.float32)
        first = jnp.where((tio == 0.0) | (te_prev != te), 1.0, 0.0)
        incl = (ia64 <= ib64).astype(jnp.float32)
        runidx = jnp.dot(first, incl, preferred_element_type=jnp.float32)
        rm1 = runidx - 1.0
        par = rm1 - 2.0 * jnp.floor(rm1 * 0.5)
        nxt = jnp.full((1, 64), float(E + 1), jnp.float32)
        for e in range(E, -1, -1):
            if e == E:
                pres = jnp.full((1, 1), 1.0, jnp.float32)
            else:
                pres = (ptiles[:, e:e + 1] > 0.0).astype(jnp.float32)
            take = (te < float(e)) & (pres > 0.0)
            nxt = jnp.where(take, float(e), nxt)
        first_ref[...] = first.astype(jnp.int32)
        par_ref[...] = par.astype(jnp.int32)
        nxt_ref[...] = nxt.astype(jnp.int32)


def _routing(xf, Wg):
    out_shapes = (
        jax.ShapeDtypeStruct((N, E), jnp.float32),
        jax.ShapeDtypeStruct((1, E), jnp.float32),
        jax.ShapeDtypeStruct((1, 64), jnp.int32),
        jax.ShapeDtypeStruct((1, 64), jnp.int32),
        jax.ShapeDtypeStruct((1, 64), jnp.int32),
        jax.ShapeDtypeStruct((1, 64), jnp.int32),
    )
    small_spec = pl.BlockSpec((1, 64), lambda i: (0, 0))
    return pl.pallas_call(
        _routing_body,
        grid=(NTILES_R,),
        in_specs=[pl.BlockSpec((TMR, D), lambda i: (i, 0)),
                  pl.BlockSpec((D, E), lambda i: (0, 0))],
        out_specs=(pl.BlockSpec((TMR, E), lambda i: (i, 0)),
                   pl.BlockSpec((1, E), lambda i: (0, 0)),
                   small_spec, small_spec, small_spec, small_spec),
        out_shape=out_shapes,
        scratch_shapes=[pltpu.VMEM((1, E), jnp.float32)],
    )(xf, Wg)


def _dispatch_body(x_hbm, e0_hbm, e1_hbm, r0_hbm, r1_hbm, poff_hbm,
                   xs_hbm, pos0_hbm, pos1_hbm,
                   poff_v, e0v, r0v, e1v, r1v, idx0_v, idx1_v, rows_v,
                   semL, semR, semS):
    wid = lax.axis_index("s") * NC + lax.axis_index("c")
    base = wid * TPW
    rows_d = pltpu.async_copy(x_hbm.at[pl.ds(base, TPW)], rows_v, semR)
    d1 = pltpu.async_copy(e0_hbm.at[pl.ds(base, TPW)], e0v, semL)
    d2 = pltpu.async_copy(r0_hbm.at[pl.ds(base, TPW)], r0v, semL)
    d3 = pltpu.async_copy(e1_hbm.at[pl.ds(base, TPW)], e1v, semL)
    d4 = pltpu.async_copy(r1_hbm.at[pl.ds(base, TPW)], r1v, semL)
    d5 = pltpu.async_copy(poff_hbm, poff_v, semL)
    d1.wait(); d2.wait(); d3.wait(); d4.wait(); d5.wait()
    for j in range(TPW // 16):
        sl = pl.ds(16 * j, 16)
        idx0_v[sl] = plsc.load_gather(poff_v, [e0v[sl]]) + r0v[sl]
        idx1_v[sl] = plsc.load_gather(poff_v, [e1v[sl]]) + r1v[sl]
    rows_d.wait()
    s0 = pltpu.async_copy(rows_v, xs_hbm.at[idx0_v], semS)
    s1 = pltpu.async_copy(rows_v, xs_hbm.at[idx1_v], semS)
    s2 = pltpu.async_copy(rows_v, xs_hbm.at[pl.ds(CAP + base, TPW)], semS)
    s3 = pltpu.async_copy(idx0_v, pos0_hbm.at[pl.ds(base, TPW)], semS)
    s4 = pltpu.async_copy(idx1_v, pos1_hbm.at[pl.ds(base, TPW)], semS)
    s0.wait(); s1.wait(); s2.wait(); s3.wait(); s4.wait()


def _dispatch(xf, e0i, e1i, r0i, r1i, poffi):
    mesh = plsc.VectorSubcoreMesh(core_axis_name="c", subcore_axis_name="s")
    f = functools.partial(
        pl.kernel,
        out_type=(jax.ShapeDtypeStruct((XROWS, D), jnp.float32),
                  jax.ShapeDtypeStruct((N,), jnp.int32),
                  jax.ShapeDtypeStruct((N,), jnp.int32)),
        mesh=mesh,
        scratch_types=[pltpu.VMEM((16,), jnp.int32),
                       pltpu.VMEM((TPW,), jnp.int32),
                       pltpu.VMEM((TPW,), jnp.int32),
                       pltpu.VMEM((TPW,), jnp.int32),
                       pltpu.VMEM((TPW,), jnp.int32),
                       pltpu.VMEM((TPW,), jnp.int32),
                       pltpu.VMEM((TPW,), jnp.int32),
                       pltpu.VMEM((TPW, D), jnp.float32),
                       pltpu.SemaphoreType.DMA,
                       pltpu.SemaphoreType.DMA,
                       pltpu.SemaphoreType.DMA],
        compiler_params=pltpu.CompilerParams(needs_layout_passes=False),
    )(_dispatch_body)
    return f(xf, e0i, e1i, r0i, r1i, poffi)


def _issue_load(e, b, we1, we2, ws1, ws2, w1buf, w2buf, sem1, sem2):
    @pl.when(e < E)
    def _():
        pltpu.make_async_copy(we1.at[e], w1buf.at[b], sem1.at[b]).start()
        pltpu.make_async_copy(we2.at[e], w2buf.at[b], sem2.at[b]).start()

    @pl.when(e == E)
    def _():
        pltpu.make_async_copy(ws1, w1buf.at[b], sem1.at[b]).start()
        pltpu.make_async_copy(ws2, w2buf.at[b], sem2.at[b]).start()


def _gmm_body(te_s, first_s, par_s, nxt_s, xs_ref, we1, we2, ws1, ws2,
              ys_ref, w1buf, w2buf, sem1, sem2):
    t = pl.program_id(0)

    @pl.when(t == 0)
    def _():
        _issue_load(te_s[0], 0, we1, we2, ws1, ws2, w1buf, w2buf, sem1, sem2)

    @pl.when(first_s[t] == 1)
    def _():
        b = par_s[t]
        pltpu.make_async_copy(we1.at[0], w1buf.at[b], sem1.at[b]).wait()
        pltpu.make_async_copy(we2.at[0], w2buf.at[b], sem2.at[b]).wait()
        _issue_load(nxt_s[t], 1 - b, we1, we2, ws1, ws2, w1buf, w2buf,
                    sem1, sem2)

    b = par_s[t]
    h = jax.nn.silu(jnp.dot(xs_ref[...], w1buf[b],
                            preferred_element_type=jnp.float32))
    ys_ref[...] = jnp.dot(h, w2buf[b], preferred_element_type=jnp.float32)


def _gmm(te, first, par, nxt, xs, We1, We2, Ws1, Ws2):
    grid_spec = pltpu.PrefetchScalarGridSpec(
        num_scalar_prefetch=4,
        grid=(NTILES_ALL,),
        in_specs=[pl.BlockSpec((TM, D), lambda t, *_: (t, 0)),
                  pl.BlockSpec(memory_space=pl.ANY),
                  pl.BlockSpec(memory_space=pl.ANY),
                  pl.BlockSpec(memory_space=pl.ANY),
                  pl.BlockSpec(memory_space=pl.ANY)],
        out_specs=pl.BlockSpec((TM, D), lambda t, *_: (t, 0)),
        scratch_shapes=[pltpu.VMEM((2, D, H), jnp.float32),
                        pltpu.VMEM((2, H, D), jnp.float32),
                        pltpu.SemaphoreType.DMA((2,)),
                        pltpu.SemaphoreType.DMA((2,))],
    )
    return pl.pallas_call(
        _gmm_body,
        grid_spec=grid_spec,
        out_shape=jax.ShapeDtypeStruct((XROWS, D), jnp.float32),
        compiler_params=pltpu.CompilerParams(
            vmem_limit_bytes=100 * 1024 * 1024),
    )(te, first, par, nxt, xs, We1, We2, Ws1, Ws2)


def _combine_body(ys_hbm, pos0_hbm, pos1_hbm, g0_hbm, g1_hbm,
                  out_hbm,
                  idx0_v, idx1_v, g0_v, g1_v, y0_v, y1_v, s_v, o_v,
                  semL, semW0, semW1):
    semW = (semW0, semW1)
    wid = lax.axis_index("s") * NC + lax.axis_index("c")
    base = wid * TPW
    d1 = pltpu.async_copy(pos0_hbm.at[pl.ds(base, TPW)], idx0_v, semL)
    d2 = pltpu.async_copy(pos1_hbm.at[pl.ds(base, TPW)], idx1_v, semL)
    d3 = pltpu.async_copy(g0_hbm.at[pl.ds(base, TPW)], g0_v, semL)
    d4 = pltpu.async_copy(g1_hbm.at[pl.ds(base, TPW)], g1_v, semL)
    d1.wait(); d2.wait(); d3.wait(); d4.wait()

    def issue(c):
        p = c % 2
        sl = pl.ds(c * CHUNK, CHUNK)
        return (
            pltpu.async_copy(ys_hbm.at[idx0_v.at[sl]], y0_v.at[p], semL),
            pltpu.async_copy(ys_hbm.at[idx1_v.at[sl]], y1_v.at[p], semL),
            pltpu.async_copy(ys_hbm.at[pl.ds(CAP + base + c * CHUNK, CHUNK)],
                             s_v.at[p], semL),
        )

    descs = issue(0)
    wdescs = [None, None]
    for c in range(NCHUNK):
        p = c % 2
        for dd in descs:
            dd.wait()
        if c + 1 < NCHUNK:
            descs = issue(c + 1)
        if wdescs[p] is not None:
            wdescs[p].wait()

        def token_body(j, _):
            jj = jnp.full((16,), c * CHUNK + j, jnp.int32)
            gj0 = plsc.load_gather(g0_v, [jj])
            gj1 = plsc.load_gather(g1_v, [jj])
            for k in range(D // 16):
                sl = pl.ds(16 * k, 16)
                o_v[p, j, sl] = (s_v[p, j, sl] + gj0 * y0_v[p, j, sl]
                                 + gj1 * y1_v[p, j, sl])
            return 0

        lax.fori_loop(0, CHUNK, token_body, 0)
        wdescs[p] = pltpu.async_copy(
            o_v.at[p], out_hbm.at[pl.ds(base + c * CHUNK, CHUNK)], semW[p])
    for wd in wdescs:
        if wd is not None:
            wd.wait()


def _combine(ys, pos0, pos1, g0, g1):
    mesh = plsc.VectorSubcoreMesh(core_axis_name="c", subcore_axis_name="s")
    f = functools.partial(
        pl.kernel,
        out_type=jax.ShapeDtypeStruct((N, D), jnp.float32),
        mesh=mesh,
        scratch_types=[pltpu.VMEM((TPW,), jnp.int32),
                       pltpu.VMEM((TPW,), jnp.int32),
                       pltpu.VMEM((TPW,), jnp.float32),
                       pltpu.VMEM((TPW,), jnp.float32),
                       pltpu.VMEM((2, CHUNK, D), jnp.float32),
                       pltpu.VMEM((2, CHUNK, D), jnp.float32),
                       pltpu.VMEM((2, CHUNK, D), jnp.float32),
                       pltpu.VMEM((2, CHUNK, D), jnp.float32),
                       pltpu.SemaphoreType.DMA,
                       pltpu.SemaphoreType.DMA,
                       pltpu.SemaphoreType.DMA],
        compiler_params=pltpu.CompilerParams(needs_layout_passes=False),
    )(_combine_body)
    return f(ys, pos0, pos1, g0, g1)


def kernel(x, Ws1, Ws2, We1, We2, Wg):
    Bn, Sn, Dn = x.shape
    xf = x.reshape(N, D)
    pk, poff, te, first, par, nxt = _routing(xf, Wg)
    e0i = pk[:, 0].astype(jnp.int32)
    e1i = pk[:, 1].astype(jnp.int32)
    g0 = pk[:, 2]
    g1 = pk[:, 3]
    r0i = pk[:, 4].astype(jnp.int32)
    r1i = pk[:, 5].astype(jnp.int32)
    poffi = jnp.pad(poff.reshape(E).astype(jnp.int32), (0, 16 - E))

    xs, pos0, pos1 = _dispatch(xf, e0i, e1i, r0i, r1i, poffi)
    ys = _gmm(te.reshape(64), first.reshape(64), par.reshape(64),
              nxt.reshape(64), xs, We1, We2, Ws1, Ws2)
    out = _combine(ys, pos0, pos1, g0, g1)
    return out.reshape(Bn, Sn, Dn)


# speedup vs baseline: 1.2083x; 1.0055x over previous
import functools

import jax
import jax.numpy as jnp
from jax import lax
from jax.experimental import pallas as pl
from jax.experimental.pallas import tpu as pltpu
from jax.experimental.pallas import tpu_sc as plsc

N = 2048
D = 768
E = 8
H = 3072
TM = 128
NTOK_TILES = N // TM
NTILES_MOE = N * 2 // TM + 8
CAP = NTILES_MOE * TM
NTILES_ALL = NTILES_MOE + NTOK_TILES
XROWS = CAP + N
NC = 2
NS = 16
NW = NC * NS
TPW = N // NW
CHUNK = 16
NCHUNK = TPW // CHUNK
TMR = 512
NTILES_R = N // TMR


def _routing_body(x_ref, wg_ref, pk_ref, poff_ref, te_ref, first_ref,
                  par_ref, nxt_ref, carry):
    i = pl.program_id(0)

    @pl.when(i == 0)
    def _():
        carry[...] = jnp.zeros_like(carry)

    logits = jnp.dot(x_ref[...], wg_ref[...],
                     preferred_element_type=jnp.float32)
    iota_e = lax.broadcasted_iota(jnp.int32, (TMR, E), 1)
    m0 = jnp.max(logits, axis=1, keepdims=True)
    e0 = jnp.min(jnp.where(logits == m0, iota_e, E), axis=1, keepdims=True)
    oh0 = (iota_e == e0).astype(jnp.float32)
    masked = jnp.where(iota_e == e0, -jnp.inf, logits)
    m1 = jnp.max(masked, axis=1, keepdims=True)
    e1 = jnp.min(jnp.where(masked == m1, iota_e, E), axis=1, keepdims=True)
    oh1 = (iota_e == e1).astype(jnp.float32)
    g0 = 1.0 / (1.0 + jnp.exp(m1 - m0))

    tot = oh0 + oh1
    ir = lax.broadcasted_iota(jnp.int32, (TMR, TMR), 0)
    ic = lax.broadcasted_iota(jnp.int32, (TMR, TMR), 1)
    tri = (ic < ir).astype(jnp.float32)
    cum = carry[...] + jnp.dot(tri, tot, preferred_element_type=jnp.float32)
    r0 = jnp.sum(cum * oh0, axis=1, keepdims=True)
    r1 = jnp.sum(cum * oh1, axis=1, keepdims=True)

    lane = lax.broadcasted_iota(jnp.int32, (TMR, E), 1)
    pk = jnp.where(lane == 0, e0.astype(jnp.float32), 0.0)
    pk = jnp.where(lane == 1, e1.astype(jnp.float32), pk)
    pk = jnp.where(lane == 2, g0, pk)
    pk = jnp.where(lane == 3, 1.0 - g0, pk)
    pk = jnp.where(lane == 4, r0, pk)
    pk_ref[...] = jnp.where(lane == 5, r1, pk)

    newcarry = carry[...] + jnp.sum(tot, axis=0, keepdims=True)
    carry[...] = newcarry

    @pl.when(i == NTILES_R - 1)
    def _():
        cnt = newcarry
        ptiles = jnp.floor((cnt + (TM - 1)) / TM)
        ia = lax.broadcasted_iota(jnp.int32, (E, E), 0)
        ib = lax.broadcasted_iota(jnp.int32, (E, E), 1)
        strict = (ia < ib).astype(jnp.float32)
        pcum = jnp.dot(ptiles, strict, preferred_element_type=jnp.float32)
        poff_ref[...] = pcum * TM
        pincl = pcum + ptiles
        tio = lax.broadcasted_iota(jnp.int32, (1, 64), 1).astype(jnp.float32)
        te = jnp.zeros((1, 64), jnp.float32)
        for e in range(E):
            te = te + (tio >= pincl[:, e:e + 1]).astype(jnp.float32)
        iota8 = lax.broadcasted_iota(jnp.int32, (1, E), 1).astype(jnp.float32)
        emax = jnp.max(jnp.where(ptiles > 0.0, iota8, 0.0), axis=1,
                       keepdims=True)
        te = jnp.minimum(te, emax)
        te_ref[...] = te.astype(jnp.int32)

        ia64 = lax.broadcasted_iota(jnp.int32, (64, 64), 0)
        ib64 = lax.broadcasted_iota(jnp.int32, (64, 64), 1)
        shift = (ia64 == ib64 - 1).astype(jnp.float32)
        te_prev = jnp.dot(te, shift, preferred_element_type=jnp.float32)
        first = jnp.where((tio == 0.0) | (te_prev != te), 1.0, 0.0)
        incl = (ia64 <= ib64).astype(jnp.float32)
        runidx = jnp.dot(first, incl, preferred_element_type=jnp.float32)
        rm1 = runidx - 1.0
        par = rm1 - 2.0 * jnp.floor(rm1 * 0.5)
        nxt = jnp.full((1, 64), float(E + 1), jnp.float32)
        for e in range(E - 1, -1, -1):
            pres = (ptiles[:, e:e + 1] > 0.0).astype(jnp.float32)
            take = (te < float(e)) & (pres > 0.0)
            nxt = jnp.where(take, float(e), nxt)
        first_ref[...] = first.astype(jnp.int32)
        par_ref[...] = par.astype(jnp.int32)
        nxt_ref[...] = nxt.astype(jnp.int32)


def _routing(xf, Wg):
    out_shapes = (
        jax.ShapeDtypeStruct((N, E), jnp.float32),
        jax.ShapeDtypeStruct((1, E), jnp.float32),
        jax.ShapeDtypeStruct((1, 64), jnp.int32),
        jax.ShapeDtypeStruct((1, 64), jnp.int32),
        jax.ShapeDtypeStruct((1, 64), jnp.int32),
        jax.ShapeDtypeStruct((1, 64), jnp.int32),
    )
    small_spec = pl.BlockSpec((1, 64), lambda i: (0, 0))
    return pl.pallas_call(
        _routing_body,
        grid=(NTILES_R,),
        in_specs=[pl.BlockSpec((TMR, D), lambda i: (i, 0)),
                  pl.BlockSpec((D, E), lambda i: (0, 0))],
        out_specs=(pl.BlockSpec((TMR, E), lambda i: (i, 0)),
                   pl.BlockSpec((1, E), lambda i: (0, 0)),
                   small_spec, small_spec, small_spec, small_spec),
        out_shape=out_shapes,
        scratch_shapes=[pltpu.VMEM((1, E), jnp.float32)],
    )(xf, Wg)


def _dispatch_body(x_hbm, e0_hbm, e1_hbm, r0_hbm, r1_hbm, poff_hbm,
                   xs_hbm, pos0_hbm, pos1_hbm,
                   poff_v, e0v, r0v, e1v, r1v, idx0_v, idx1_v, rows_v,
                   semL, semR, semS):
    wid = lax.axis_index("s") * NC + lax.axis_index("c")
    base = wid * TPW
    rows_d = pltpu.async_copy(x_hbm.at[pl.ds(base, TPW)], rows_v, semR)
    d1 = pltpu.async_copy(e0_hbm.at[pl.ds(base, TPW)], e0v, semL)
    d2 = pltpu.async_copy(r0_hbm.at[pl.ds(base, TPW)], r0v, semL)
    d3 = pltpu.async_copy(e1_hbm.at[pl.ds(base, TPW)], e1v, semL)
    d4 = pltpu.async_copy(r1_hbm.at[pl.ds(base, TPW)], r1v, semL)
    d5 = pltpu.async_copy(poff_hbm, poff_v, semL)
    d1.wait(); d2.wait(); d3.wait(); d4.wait(); d5.wait()
    for j in range(TPW // 16):
        sl = pl.ds(16 * j, 16)
        idx0_v[sl] = plsc.load_gather(poff_v, [e0v[sl]]) + r0v[sl]
        idx1_v[sl] = plsc.load_gather(poff_v, [e1v[sl]]) + r1v[sl]
    rows_d.wait()
    s0 = pltpu.async_copy(rows_v, xs_hbm.at[idx0_v], semS)
    s1 = pltpu.async_copy(rows_v, xs_hbm.at[idx1_v], semS)
    s3 = pltpu.async_copy(idx0_v, pos0_hbm.at[pl.ds(base, TPW)], semS)
    s4 = pltpu.async_copy(idx1_v, pos1_hbm.at[pl.ds(base, TPW)], semS)
    s0.wait(); s1.wait(); s3.wait(); s4.wait()


def _dispatch(xf, e0i, e1i, r0i, r1i, poffi):
    mesh = plsc.VectorSubcoreMesh(core_axis_name="c", subcore_axis_name="s")
    f = functools.partial(
        pl.kernel,
        out_type=(jax.ShapeDtypeStruct((CAP, D), jnp.float32),
                  jax.ShapeDtypeStruct((N,), jnp.int32),
                  jax.ShapeDtypeStruct((N,), jnp.int32)),
        mesh=mesh,
        scratch_types=[pltpu.VMEM((16,), jnp.int32),
                       pltpu.VMEM((TPW,), jnp.int32),
                       pltpu.VMEM((TPW,), jnp.int32),
                       pltpu.VMEM((TPW,), jnp.int32),
                       pltpu.VMEM((TPW,), jnp.int32),
                       pltpu.VMEM((TPW,), jnp.int32),
                       pltpu.VMEM((TPW,), jnp.int32),
                       pltpu.VMEM((TPW, D), jnp.float32),
                       pltpu.SemaphoreType.DMA,
                       pltpu.SemaphoreType.DMA,
                       pltpu.SemaphoreType.DMA],
        compiler_params=pltpu.CompilerParams(needs_layout_passes=False),
    )(_dispatch_body)
    return f(xf, e0i, e1i, r0i, r1i, poffi)


def _issue_load(e, b, we1, we2, w1buf, w2buf, sem1, sem2):
    @pl.when(e < E)
    def _():
        pltpu.make_async_copy(we1.at[e], w1buf.at[b], sem1.at[b]).start()
        pltpu.make_async_copy(we2.at[e], w2buf.at[b], sem2.at[b]).start()


def _gmm_body(te_s, first_s, par_s, nxt_s, xs_ref, we1, we2,
              ys_ref, w1buf, w2buf, sem1, sem2):
    t = pl.program_id(0)

    @pl.when(t == 0)
    def _():
        _issue_load(te_s[0], 0, we1, we2, w1buf, w2buf, sem1, sem2)

    @pl.when(first_s[t] == 1)
    def _():
        b = par_s[t]
        pltpu.make_async_copy(we1.at[0], w1buf.at[b], sem1.at[b]).wait()
        pltpu.make_async_copy(we2.at[0], w2buf.at[b], sem2.at[b]).wait()
        _issue_load(nxt_s[t], 1 - b, we1, we2, w1buf, w2buf, sem1, sem2)

    b = par_s[t]
    h = jax.nn.silu(jnp.dot(xs_ref[...], w1buf[b],
                            preferred_element_type=jnp.float32))
    ys_ref[...] = jnp.dot(h, w2buf[b], preferred_element_type=jnp.float32)


def _gmm(te, first, par, nxt, xs, We1, We2):
    grid_spec = pltpu.PrefetchScalarGridSpec(
        num_scalar_prefetch=4,
        grid=(NTILES_MOE,),
        in_specs=[pl.BlockSpec((TM, D), lambda t, *_: (t, 0)),
                  pl.BlockSpec(memory_space=pl.ANY),
                  pl.BlockSpec(memory_space=pl.ANY)],
        out_specs=pl.BlockSpec((TM, D), lambda t, *_: (t, 0)),
        scratch_shapes=[pltpu.VMEM((2, D, H), jnp.float32),
                        pltpu.VMEM((2, H, D), jnp.float32),
                        pltpu.SemaphoreType.DMA((2,)),
                        pltpu.SemaphoreType.DMA((2,))],
    )
    return pl.pallas_call(
        _gmm_body,
        grid_spec=grid_spec,
        out_shape=jax.ShapeDtypeStruct((CAP, D), jnp.float32),
        compiler_params=pltpu.CompilerParams(
            vmem_limit_bytes=100 * 1024 * 1024),
    )(te, first, par, nxt, xs, We1, We2)


def _shared_body(x_ref, w1_ref, w2_ref, o_ref):
    h = jax.nn.silu(jnp.dot(x_ref[...], w1_ref[...],
                            preferred_element_type=jnp.float32))
    o_ref[...] = jnp.dot(h, w2_ref[...], preferred_element_type=jnp.float32)


def _shared(xf, Ws1, Ws2):
    return pl.pallas_call(
        _shared_body,
        grid=(NTOK_TILES,),
        in_specs=[pl.BlockSpec((TM, D), lambda i: (i, 0)),
                  pl.BlockSpec((D, H), lambda i: (0, 0)),
                  pl.BlockSpec((H, D), lambda i: (0, 0))],
        out_specs=pl.BlockSpec((TM, D), lambda i: (i, 0)),
        out_shape=jax.ShapeDtypeStruct((N, D), jnp.float32),
    )(xf, Ws1, Ws2)


def _combine_body(ys_hbm, sh_hbm, pos0_hbm, pos1_hbm, g0_hbm, g1_hbm,
                  out_hbm,
                  idx0_v, idx1_v, g0_v, g1_v, y0_v, y1_v, s_v, o_v,
                  semL, semW0, semW1):
    semW = (semW0, semW1)
    wid = lax.axis_index("s") * NC + lax.axis_index("c")
    base = wid * TPW
    d1 = pltpu.async_copy(pos0_hbm.at[pl.ds(base, TPW)], idx0_v, semL)
    d2 = pltpu.async_copy(pos1_hbm.at[pl.ds(base, TPW)], idx1_v, semL)
    d3 = pltpu.async_copy(g0_hbm.at[pl.ds(base, TPW)], g0_v, semL)
    d4 = pltpu.async_copy(g1_hbm.at[pl.ds(base, TPW)], g1_v, semL)
    d1.wait(); d2.wait(); d3.wait(); d4.wait()

    def issue(c):
        p = c % 2
        sl = pl.ds(c * CHUNK, CHUNK)
        return (
            pltpu.async_copy(ys_hbm.at[idx0_v.at[sl]], y0_v.at[p], semL),
            pltpu.async_copy(ys_hbm.at[idx1_v.at[sl]], y1_v.at[p], semL),
            pltpu.async_copy(sh_hbm.at[pl.ds(base + c * CHUNK, CHUNK)],
                             s_v.at[p], semL),
        )

    descs = issue(0)
    wdescs = [None, None]
    for c in range(NCHUNK):
        p = c % 2
        for dd in descs:
            dd.wait()
        if c + 1 < NCHUNK:
            descs = issue(c + 1)
        if wdescs[p] is not None:
            wdescs[p].wait()

        def token_body(j, _):
            jj = jnp.full((16,), c * CHUNK + j, jnp.int32)
            gj0 = plsc.load_gather(g0_v, [jj])
            gj1 = plsc.load_gather(g1_v, [jj])
            for k in range(D // 16):
                sl = pl.ds(16 * k, 16)
                o_v[p, j, sl] = (s_v[p, j, sl] + gj0 * y0_v[p, j, sl]
                                 + gj1 * y1_v[p, j, sl])
            return 0

        lax.fori_loop(0, CHUNK, token_body, 0)
        wdescs[p] = pltpu.async_copy(
            o_v.at[p], out_hbm.at[pl.ds(base + c * CHUNK, CHUNK)], semW[p])
    for wd in wdescs:
        if wd is not None:
            wd.wait()


def _combine(ys, sh, pos0, pos1, g0, g1):
    mesh = plsc.VectorSubcoreMesh(core_axis_name="c", subcore_axis_name="s")
    f = functools.partial(
        pl.kernel,
        out_type=jax.ShapeDtypeStruct((N, D), jnp.float32),
        mesh=mesh,
        scratch_types=[pltpu.VMEM((TPW,), jnp.int32),
                       pltpu.VMEM((TPW,), jnp.int32),
                       pltpu.VMEM((TPW,), jnp.float32),
                       pltpu.VMEM((TPW,), jnp.float32),
                       pltpu.VMEM((2, CHUNK, D), jnp.float32),
                       pltpu.VMEM((2, CHUNK, D), jnp.float32),
                       pltpu.VMEM((2, CHUNK, D), jnp.float32),
                       pltpu.VMEM((2, CHUNK, D), jnp.float32),
                       pltpu.SemaphoreType.DMA,
                       pltpu.SemaphoreType.DMA,
                       pltpu.SemaphoreType.DMA],
        compiler_params=pltpu.CompilerParams(needs_layout_passes=False),
    )(_combine_body)
    return f(ys, sh, pos0, pos1, g0, g1)


def kernel(x, Ws1, Ws2, We1, We2, Wg):
    Bn, Sn, Dn = x.shape
    xf = x.reshape(N, D)
    pk, poff, te, first, par, nxt = _routing(xf, Wg)
    e0i = pk[:, 0].astype(jnp.int32)
    e1i = pk[:, 1].astype(jnp.int32)
    g0 = pk[:, 2]
    g1 = pk[:, 3]
    r0i = pk[:, 4].astype(jnp.int32)
    r1i = pk[:, 5].astype(jnp.int32)
    poffi = jnp.pad(poff.reshape(E).astype(jnp.int32), (0, 16 - E))

    xs, pos0, pos1 = _dispatch(xf, e0i, e1i, r0i, r1i, poffi)
    sh = _shared(xf, Ws1, Ws2)
    ys = _gmm(te.reshape(64), first.reshape(64), par.reshape(64),
              nxt.reshape(64), xs, We1, We2)
    out = _combine(ys, sh, pos0, pos1, g0, g1)
    return out.reshape(Bn, Sn, Dn)


# speedup vs baseline: 1.2450x; 1.0304x over previous
import functools

import jax
import jax.numpy as jnp
from jax import lax
from jax.experimental import pallas as pl
from jax.experimental.pallas import tpu as pltpu
from jax.experimental.pallas import tpu_sc as plsc

N = 2048
D = 768
E = 8
H = 3072
TM = 128
NTOK_TILES = N // TM
NTILES_MOE = N * 2 // TM + 8
CAP = NTILES_MOE * TM
NTILES_ALL = NTILES_MOE + NTOK_TILES
XROWS = CAP + N
NC = 2
NS = 16
NW = NC * NS
TPW = N // NW
CHUNK = 16
NCHUNK = TPW // CHUNK
TMR = 512
NTILES_R = N // TMR


def _routing_body(x_ref, wg_ref, pk_ref, poff_ref, te_ref, first_ref,
                  par_ref, nxt_ref, carry):
    i = pl.program_id(0)

    @pl.when(i == 0)
    def _():
        carry[...] = jnp.zeros_like(carry)

    logits = jnp.dot(x_ref[...], wg_ref[...],
                     preferred_element_type=jnp.float32)
    iota_e = lax.broadcasted_iota(jnp.int32, (TMR, E), 1)
    m0 = jnp.max(logits, axis=1, keepdims=True)
    e0 = jnp.min(jnp.where(logits == m0, iota_e, E), axis=1, keepdims=True)
    oh0 = (iota_e == e0).astype(jnp.float32)
    masked = jnp.where(iota_e == e0, -jnp.inf, logits)
    m1 = jnp.max(masked, axis=1, keepdims=True)
    e1 = jnp.min(jnp.where(masked == m1, iota_e, E), axis=1, keepdims=True)
    oh1 = (iota_e == e1).astype(jnp.float32)
    g0 = 1.0 / (1.0 + jnp.exp(m1 - m0))

    tot = oh0 + oh1
    ir = lax.broadcasted_iota(jnp.int32, (TMR, TMR), 0)
    ic = lax.broadcasted_iota(jnp.int32, (TMR, TMR), 1)
    tri = (ic < ir).astype(jnp.float32)
    cum = carry[...] + jnp.dot(tri, tot, preferred_element_type=jnp.float32)
    r0 = jnp.sum(cum * oh0, axis=1, keepdims=True)
    r1 = jnp.sum(cum * oh1, axis=1, keepdims=True)

    lane = lax.broadcasted_iota(jnp.int32, (TMR, E), 1)
    pk = jnp.where(lane == 0, e0.astype(jnp.float32), 0.0)
    pk = jnp.where(lane == 1, e1.astype(jnp.float32), pk)
    pk = jnp.where(lane == 2, g0, pk)
    pk = jnp.where(lane == 3, 1.0 - g0, pk)
    pk = jnp.where(lane == 4, r0, pk)
    pk_ref[...] = jnp.where(lane == 5, r1, pk)

    newcarry = carry[...] + jnp.sum(tot, axis=0, keepdims=True)
    carry[...] = newcarry

    @pl.when(i == NTILES_R - 1)
    def _():
        cnt = newcarry
        ptiles = jnp.floor((cnt + (TM - 1)) / TM)
        ia = lax.broadcasted_iota(jnp.int32, (E, E), 0)
        ib = lax.broadcasted_iota(jnp.int32, (E, E), 1)
        strict = (ia < ib).astype(jnp.float32)
        pcum = jnp.dot(ptiles, strict, preferred_element_type=jnp.float32)
        ia8 = lax.broadcasted_iota(jnp.int32, (E, 64), 0)
        ib8 = lax.broadcasted_iota(jnp.int32, (E, 64), 1)
        spread = (ia8 == ib8).astype(jnp.float32)
        poff_ref[...] = jnp.dot(pcum * TM, spread,
                                preferred_element_type=jnp.float32
                                ).astype(jnp.int32)
        pincl = pcum + ptiles
        tio = lax.broadcasted_iota(jnp.int32, (1, 64), 1).astype(jnp.float32)
        te = jnp.zeros((1, 64), jnp.float32)
        for e in range(E):
            te = te + (tio >= pincl[:, e:e + 1]).astype(jnp.float32)
        iota8 = lax.broadcasted_iota(jnp.int32, (1, E), 1).astype(jnp.float32)
        emax = jnp.max(jnp.where(ptiles > 0.0, iota8, 0.0), axis=1,
                       keepdims=True)
        te = jnp.minimum(te, emax)
        te_ref[...] = te.astype(jnp.int32)

        ia64 = lax.broadcasted_iota(jnp.int32, (64, 64), 0)
        ib64 = lax.broadcasted_iota(jnp.int32, (64, 64), 1)
        shift = (ia64 == ib64 - 1).astype(jnp.float32)
        te_prev = jnp.dot(te, shift, preferred_element_type=jnp.float32)
        first = jnp.where((tio == 0.0) | (te_prev != te), 1.0, 0.0)
        incl = (ia64 <= ib64).astype(jnp.float32)
        runidx = jnp.dot(first, incl, preferred_element_type=jnp.float32)
        rm1 = runidx - 1.0
        par = rm1 - 2.0 * jnp.floor(rm1 * 0.5)
        nxt = jnp.full((1, 64), float(E + 1), jnp.float32)
        for e in range(E - 1, -1, -1):
            pres = (ptiles[:, e:e + 1] > 0.0).astype(jnp.float32)
            take = (te < float(e)) & (pres > 0.0)
            nxt = jnp.where(take, float(e), nxt)
        first_ref[...] = first.astype(jnp.int32)
        par_ref[...] = par.astype(jnp.int32)
        nxt_ref[...] = nxt.astype(jnp.int32)


def _routing(xf, Wg):
    out_shapes = (
        jax.ShapeDtypeStruct((N, E), jnp.float32),
        jax.ShapeDtypeStruct((1, 64), jnp.int32),
        jax.ShapeDtypeStruct((1, 64), jnp.int32),
        jax.ShapeDtypeStruct((1, 64), jnp.int32),
        jax.ShapeDtypeStruct((1, 64), jnp.int32),
        jax.ShapeDtypeStruct((1, 64), jnp.int32),
    )
    small_spec = pl.BlockSpec((1, 64), lambda i: (0, 0))
    return pl.pallas_call(
        _routing_body,
        grid=(NTILES_R,),
        in_specs=[pl.BlockSpec((TMR, D), lambda i: (i, 0)),
                  pl.BlockSpec((D, E), lambda i: (0, 0))],
        out_specs=(pl.BlockSpec((TMR, E), lambda i: (i, 0)),
                   small_spec,
                   small_spec, small_spec, small_spec, small_spec),
        out_shape=out_shapes,
        scratch_shapes=[pltpu.VMEM((1, E), jnp.float32)],
    )(xf, Wg)


def _dispatch_body(x_hbm, pk_hbm, poff_hbm,
                   xs_hbm, pos0_hbm, pos1_hbm,
                   poff_v, pk_v, idx0_v, idx1_v, rows_v,
                   semL, semR, semS):
    wid = lax.axis_index("s") * NC + lax.axis_index("c")
    base = wid * TPW
    rows_d = pltpu.async_copy(x_hbm.at[pl.ds(base, TPW)], rows_v, semR)
    d1 = pltpu.async_copy(pk_hbm.at[pl.ds(base * E, TPW * E)], pk_v, semL)
    d5 = pltpu.async_copy(poff_hbm, poff_v, semL)
    d1.wait(); d5.wait()
    for j in range(TPW // 16):
        sl = pl.ds(16 * j, 16)
        fl = lax.broadcasted_iota(jnp.int32, (16,), 0) * E + 16 * j * E
        e0 = plsc.load_gather(pk_v, [fl + 0]).astype(jnp.int32)
        e1 = plsc.load_gather(pk_v, [fl + 1]).astype(jnp.int32)
        r0 = plsc.load_gather(pk_v, [fl + 4]).astype(jnp.int32)
        r1 = plsc.load_gather(pk_v, [fl + 5]).astype(jnp.int32)
        idx0_v[sl] = plsc.load_gather(poff_v, [e0]) + r0
        idx1_v[sl] = plsc.load_gather(poff_v, [e1]) + r1
    rows_d.wait()
    s0 = pltpu.async_copy(rows_v, xs_hbm.at[idx0_v], semS)
    s1 = pltpu.async_copy(rows_v, xs_hbm.at[idx1_v], semS)
    s3 = pltpu.async_copy(idx0_v, pos0_hbm.at[pl.ds(base, TPW)], semS)
    s4 = pltpu.async_copy(idx1_v, pos1_hbm.at[pl.ds(base, TPW)], semS)
    s0.wait(); s1.wait(); s3.wait(); s4.wait()


def _dispatch(xf, pk_flat, poffi):
    mesh = plsc.VectorSubcoreMesh(core_axis_name="c", subcore_axis_name="s")
    f = functools.partial(
        pl.kernel,
        out_type=(jax.ShapeDtypeStruct((CAP, D), jnp.float32),
                  jax.ShapeDtypeStruct((N,), jnp.int32),
                  jax.ShapeDtypeStruct((N,), jnp.int32)),
        mesh=mesh,
        scratch_types=[pltpu.VMEM((64,), jnp.int32),
                       pltpu.VMEM((TPW * E,), jnp.float32),
                       pltpu.VMEM((TPW,), jnp.int32),
                       pltpu.VMEM((TPW,), jnp.int32),
                       pltpu.VMEM((TPW, D), jnp.float32),
                       pltpu.SemaphoreType.DMA,
                       pltpu.SemaphoreType.DMA,
                       pltpu.SemaphoreType.DMA],
        compiler_params=pltpu.CompilerParams(needs_layout_passes=False),
    )(_dispatch_body)
    return f(xf, pk_flat, poffi)


def _issue_load(e, b, we1, we2, w1buf, w2buf, sem1, sem2):
    @pl.when(e < E)
    def _():
        pltpu.make_async_copy(we1.at[e], w1buf.at[b], sem1.at[b]).start()
        pltpu.make_async_copy(we2.at[e], w2buf.at[b], sem2.at[b]).start()


def _gmm_body(te_s, first_s, par_s, nxt_s, xs_ref, we1, we2,
              ys_ref, w1buf, w2buf, sem1, sem2):
    t = pl.program_id(0)

    @pl.when(t == 0)
    def _():
        _issue_load(te_s[0], 0, we1, we2, w1buf, w2buf, sem1, sem2)

    @pl.when(first_s[t] == 1)
    def _():
        b = par_s[t]
        pltpu.make_async_copy(we1.at[0], w1buf.at[b], sem1.at[b]).wait()
        pltpu.make_async_copy(we2.at[0], w2buf.at[b], sem2.at[b]).wait()
        _issue_load(nxt_s[t], 1 - b, we1, we2, w1buf, w2buf, sem1, sem2)

    b = par_s[t]
    h = jax.nn.silu(jnp.dot(xs_ref[...], w1buf[b],
                            preferred_element_type=jnp.float32))
    ys_ref[...] = jnp.dot(h, w2buf[b], preferred_element_type=jnp.float32)


def _gmm(te, first, par, nxt, xs, We1, We2):
    grid_spec = pltpu.PrefetchScalarGridSpec(
        num_scalar_prefetch=4,
        grid=(NTILES_MOE,),
        in_specs=[pl.BlockSpec((TM, D), lambda t, *_: (t, 0)),
                  pl.BlockSpec(memory_space=pl.ANY),
                  pl.BlockSpec(memory_space=pl.ANY)],
        out_specs=pl.BlockSpec((TM, D), lambda t, *_: (t, 0)),
        scratch_shapes=[pltpu.VMEM((2, D, H), jnp.float32),
                        pltpu.VMEM((2, H, D), jnp.float32),
                        pltpu.SemaphoreType.DMA((2,)),
                        pltpu.SemaphoreType.DMA((2,))],
    )
    return pl.pallas_call(
        _gmm_body,
        grid_spec=grid_spec,
        out_shape=jax.ShapeDtypeStruct((CAP, D), jnp.float32),
        compiler_params=pltpu.CompilerParams(
            vmem_limit_bytes=100 * 1024 * 1024),
    )(te, first, par, nxt, xs, We1, We2)


def _shared_body(x_ref, w1_ref, w2_ref, o_ref):
    h = jax.nn.silu(jnp.dot(x_ref[...], w1_ref[...],
                            preferred_element_type=jnp.float32))
    o_ref[...] = jnp.dot(h, w2_ref[...], preferred_element_type=jnp.float32)


def _shared(xf, Ws1, Ws2):
    return pl.pallas_call(
        _shared_body,
        grid=(NTOK_TILES,),
        in_specs=[pl.BlockSpec((TM, D), lambda i: (i, 0)),
                  pl.BlockSpec((D, H), lambda i: (0, 0)),
                  pl.BlockSpec((H, D), lambda i: (0, 0))],
        out_specs=pl.BlockSpec((TM, D), lambda i: (i, 0)),
        out_shape=jax.ShapeDtypeStruct((N, D), jnp.float32),
    )(xf, Ws1, Ws2)


def _combine_body(ys_hbm, sh_hbm, pos0_hbm, pos1_hbm, pk_hbm,
                  out_hbm,
                  idx0_v, idx1_v, pk_v, y0_v, y1_v, s_v, o_v,
                  semL, semW0, semW1):
    semW = (semW0, semW1)
    wid = lax.axis_index("s") * NC + lax.axis_index("c")
    base = wid * TPW
    d1 = pltpu.async_copy(pos0_hbm.at[pl.ds(base, TPW)], idx0_v, semL)
    d2 = pltpu.async_copy(pos1_hbm.at[pl.ds(base, TPW)], idx1_v, semL)
    d3 = pltpu.async_copy(pk_hbm.at[pl.ds(base * E, TPW * E)], pk_v, semL)
    d1.wait(); d2.wait(); d3.wait()

    def issue(c):
        p = c % 2
        sl = pl.ds(c * CHUNK, CHUNK)
        return (
            pltpu.async_copy(ys_hbm.at[idx0_v.at[sl]], y0_v.at[p], semL),
            pltpu.async_copy(ys_hbm.at[idx1_v.at[sl]], y1_v.at[p], semL),
            pltpu.async_copy(sh_hbm.at[pl.ds(base + c * CHUNK, CHUNK)],
                             s_v.at[p], semL),
        )

    descs = issue(0)
    wdescs = [None, None]
    for c in range(NCHUNK):
        p = c % 2
        for dd in descs:
            dd.wait()
        if c + 1 < NCHUNK:
            descs = issue(c + 1)
        if wdescs[p] is not None:
            wdescs[p].wait()

        def token_body(j, _):
            jj = jnp.full((16,), (c * CHUNK + j) * E, jnp.int32)
            gj0 = plsc.load_gather(pk_v, [jj + 2])
            gj1 = plsc.load_gather(pk_v, [jj + 3])
            for k in range(D // 16):
                sl = pl.ds(16 * k, 16)
                o_v[p, j, sl] = (s_v[p, j, sl] + gj0 * y0_v[p, j, sl]
                                 + gj1 * y1_v[p, j, sl])
            return 0

        lax.fori_loop(0, CHUNK, token_body, 0)
        wdescs[p] = pltpu.async_copy(
            o_v.at[p], out_hbm.at[pl.ds(base + c * CHUNK, CHUNK)], semW[p])
    for wd in wdescs:
        if wd is not None:
            wd.wait()


def _combine(ys, sh, pos0, pos1, pk_flat):
    mesh = plsc.VectorSubcoreMesh(core_axis_name="c", subcore_axis_name="s")
    f = functools.partial(
        pl.kernel,
        out_type=jax.ShapeDtypeStruct((N, D), jnp.float32),
        mesh=mesh,
        scratch_types=[pltpu.VMEM((TPW,), jnp.int32),
                       pltpu.VMEM((TPW,), jnp.int32),
                       pltpu.VMEM((TPW * E,), jnp.float32),
                       pltpu.VMEM((2, CHUNK, D), jnp.float32),
                       pltpu.VMEM((2, CHUNK, D), jnp.float32),
                       pltpu.VMEM((2, CHUNK, D), jnp.float32),
                       pltpu.VMEM((2, CHUNK, D), jnp.float32),
                       pltpu.SemaphoreType.DMA,
                       pltpu.SemaphoreType.DMA,
                       pltpu.SemaphoreType.DMA],
        compiler_params=pltpu.CompilerParams(needs_layout_passes=False),
    )(_combine_body)
    return f(ys, sh, pos0, pos1, pk_flat)


def kernel(x, Ws1, Ws2, We1, We2, Wg):
    Bn, Sn, Dn = x.shape
    xf = x.reshape(N, D)
    pk, poff, te, first, par, nxt = _routing(xf, Wg)
    pk_flat = pk.reshape(N * E)

    xs, pos0, pos1 = _dispatch(xf, pk_flat, poff.reshape(64))
    sh = _shared(xf, Ws1, Ws2)
    ys = _gmm(te.reshape(64), first.reshape(64), par.reshape(64),
              nxt.reshape(64), xs, We1, We2)
    out = _combine(ys, sh, pos0, pos1, pk_flat)
    return out.reshape(Bn, Sn, Dn)


# speedup vs baseline: 1.2507x; 1.0045x over previous
import functools

import jax
import jax.numpy as jnp
from jax import lax
from jax.experimental import pallas as pl
from jax.experimental.pallas import tpu as pltpu
from jax.experimental.pallas import tpu_sc as plsc

N = 2048
D = 768
E = 8
H = 3072
TM = 128
NTOK_TILES = N // TM
NTILES_MOE = N * 2 // TM + 8
CAP = NTILES_MOE * TM
NTILES_ALL = NTILES_MOE + NTOK_TILES
XROWS = CAP + N
NC = 2
NS = 16
NW = NC * NS
TPW = N // NW
CHUNK = 16
NCHUNK = TPW // CHUNK
TMR = 512
NTILES_R = N // TMR


def _routing_body(x_ref, wg_ref, pk_ref, poff_ref, te_ref, first_ref,
                  par_ref, nxt_ref, carry):
    i = pl.program_id(0)

    @pl.when(i == 0)
    def _():
        carry[...] = jnp.zeros_like(carry)

    logits = jnp.dot(x_ref[...], wg_ref[...],
                     preferred_element_type=jnp.float32)
    iota_e = lax.broadcasted_iota(jnp.int32, (TMR, E), 1)
    m0 = jnp.max(logits, axis=1, keepdims=True)
    e0 = jnp.min(jnp.where(logits == m0, iota_e, E), axis=1, keepdims=True)
    oh0 = (iota_e == e0).astype(jnp.float32)
    masked = jnp.where(iota_e == e0, -jnp.inf, logits)
    m1 = jnp.max(masked, axis=1, keepdims=True)
    e1 = jnp.min(jnp.where(masked == m1, iota_e, E), axis=1, keepdims=True)
    oh1 = (iota_e == e1).astype(jnp.float32)
    g0 = 1.0 / (1.0 + jnp.exp(m1 - m0))

    tot = oh0 + oh1
    ir = lax.broadcasted_iota(jnp.int32, (TMR, TMR), 0)
    ic = lax.broadcasted_iota(jnp.int32, (TMR, TMR), 1)
    tri = (ic < ir).astype(jnp.float32)
    cum = carry[...] + jnp.dot(tri, tot, preferred_element_type=jnp.float32)
    r0 = jnp.sum(cum * oh0, axis=1, keepdims=True)
    r1 = jnp.sum(cum * oh1, axis=1, keepdims=True)

    lane = lax.broadcasted_iota(jnp.int32, (TMR, E), 1)
    pk = jnp.where(lane == 0, e0.astype(jnp.float32), 0.0)
    pk = jnp.where(lane == 1, e1.astype(jnp.float32), pk)
    pk = jnp.where(lane == 2, g0, pk)
    pk = jnp.where(lane == 3, 1.0 - g0, pk)
    pk = jnp.where(lane == 4, r0, pk)
    pk_ref[...] = jnp.where(lane == 5, r1, pk)

    newcarry = carry[...] + jnp.sum(tot, axis=0, keepdims=True)
    carry[...] = newcarry

    @pl.when(i == NTILES_R - 1)
    def _():
        cnt = newcarry
        ptiles = jnp.floor((cnt + (TM - 1)) / TM)
        ia = lax.broadcasted_iota(jnp.int32, (E, E), 0)
        ib = lax.broadcasted_iota(jnp.int32, (E, E), 1)
        strict = (ia < ib).astype(jnp.float32)
        pcum = jnp.dot(ptiles, strict, preferred_element_type=jnp.float32)
        ia8 = lax.broadcasted_iota(jnp.int32, (E, 64), 0)
        ib8 = lax.broadcasted_iota(jnp.int32, (E, 64), 1)
        spread = (ia8 == ib8).astype(jnp.float32)
        poff_ref[...] = jnp.dot(pcum * TM, spread,
                                preferred_element_type=jnp.float32
                                ).astype(jnp.int32)
        pincl = pcum + ptiles
        tio = lax.broadcasted_iota(jnp.int32, (1, 64), 1).astype(jnp.float32)
        te = jnp.zeros((1, 64), jnp.float32)
        for e in range(E):
            te = te + (tio >= pincl[:, e:e + 1]).astype(jnp.float32)
        iota8 = lax.broadcasted_iota(jnp.int32, (1, E), 1).astype(jnp.float32)
        emax = jnp.max(jnp.where(ptiles > 0.0, iota8, 0.0), axis=1,
                       keepdims=True)
        te = jnp.minimum(te, emax)
        te_ref[...] = te.astype(jnp.int32)

        ia64 = lax.broadcasted_iota(jnp.int32, (64, 64), 0)
        ib64 = lax.broadcasted_iota(jnp.int32, (64, 64), 1)
        shift = (ia64 == ib64 - 1).astype(jnp.float32)
        te_prev = jnp.dot(te, shift, preferred_element_type=jnp.float32)
        first = jnp.where((tio == 0.0) | (te_prev != te), 1.0, 0.0)
        incl = (ia64 <= ib64).astype(jnp.float32)
        runidx = jnp.dot(first, incl, preferred_element_type=jnp.float32)
        rm1 = runidx - 1.0
        par = rm1 - 2.0 * jnp.floor(rm1 * 0.5)
        nxt = jnp.full((1, 64), float(E + 1), jnp.float32)
        for e in range(E - 1, -1, -1):
            pres = (ptiles[:, e:e + 1] > 0.0).astype(jnp.float32)
            take = (te < float(e)) & (pres > 0.0)
            nxt = jnp.where(take, float(e), nxt)
        first_ref[...] = first.astype(jnp.int32)
        par_ref[...] = par.astype(jnp.int32)
        nxt_ref[...] = nxt.astype(jnp.int32)


def _routing(xf, Wg):
    out_shapes = (
        jax.ShapeDtypeStruct((N, E), jnp.float32),
        jax.ShapeDtypeStruct((1, 64), jnp.int32),
        jax.ShapeDtypeStruct((1, 64), jnp.int32),
        jax.ShapeDtypeStruct((1, 64), jnp.int32),
        jax.ShapeDtypeStruct((1, 64), jnp.int32),
        jax.ShapeDtypeStruct((1, 64), jnp.int32),
    )
    small_spec = pl.BlockSpec((1, 64), lambda i: (0, 0))
    return pl.pallas_call(
        _routing_body,
        grid=(NTILES_R,),
        in_specs=[pl.BlockSpec((TMR, D), lambda i: (i, 0)),
                  pl.BlockSpec((D, E), lambda i: (0, 0))],
        out_specs=(pl.BlockSpec((TMR, E), lambda i: (i, 0)),
                   small_spec,
                   small_spec, small_spec, small_spec, small_spec),
        out_shape=out_shapes,
        scratch_shapes=[pltpu.VMEM((1, E), jnp.float32)],
    )(xf, Wg)


def _dispatch_body(x_hbm, pk_hbm, poff_hbm,
                   xs_hbm, pos0_hbm, pos1_hbm,
                   poff_v, pk_v, idx0_v, idx1_v, rows_v,
                   semL, semR, semS):
    wid = lax.axis_index("s") * NC + lax.axis_index("c")
    base = wid * TPW
    rows_d = pltpu.async_copy(x_hbm.at[pl.ds(base, TPW)], rows_v, semR)
    d1 = pltpu.async_copy(pk_hbm.at[pl.ds(base * E, TPW * E)], pk_v, semL)
    d5 = pltpu.async_copy(poff_hbm, poff_v, semL)
    d1.wait(); d5.wait()
    for j in range(TPW // 16):
        sl = pl.ds(16 * j, 16)
        fl = lax.broadcasted_iota(jnp.int32, (16,), 0) * E + 16 * j * E
        e0 = plsc.load_gather(pk_v, [fl + 0]).astype(jnp.int32)
        e1 = plsc.load_gather(pk_v, [fl + 1]).astype(jnp.int32)
        r0 = plsc.load_gather(pk_v, [fl + 4]).astype(jnp.int32)
        r1 = plsc.load_gather(pk_v, [fl + 5]).astype(jnp.int32)
        idx0_v[sl] = plsc.load_gather(poff_v, [e0]) + r0
        idx1_v[sl] = plsc.load_gather(poff_v, [e1]) + r1
    rows_d.wait()
    s0 = pltpu.async_copy(rows_v, xs_hbm.at[idx0_v], semS)
    s1 = pltpu.async_copy(rows_v, xs_hbm.at[idx1_v], semS)
    s3 = pltpu.async_copy(idx0_v, pos0_hbm.at[pl.ds(base, TPW)], semS)
    s4 = pltpu.async_copy(idx1_v, pos1_hbm.at[pl.ds(base, TPW)], semS)
    s0.wait(); s1.wait(); s3.wait(); s4.wait()


def _dispatch(xf, pk_flat, poffi):
    mesh = plsc.VectorSubcoreMesh(core_axis_name="c", subcore_axis_name="s")
    f = functools.partial(
        pl.kernel,
        out_type=(jax.ShapeDtypeStruct((CAP, D), jnp.float32),
                  jax.ShapeDtypeStruct((N,), jnp.int32),
                  jax.ShapeDtypeStruct((N,), jnp.int32)),
        mesh=mesh,
        scratch_types=[pltpu.VMEM((64,), jnp.int32),
                       pltpu.VMEM((TPW * E,), jnp.float32),
                       pltpu.VMEM((TPW,), jnp.int32),
                       pltpu.VMEM((TPW,), jnp.int32),
                       pltpu.VMEM((TPW, D), jnp.float32),
                       pltpu.SemaphoreType.DMA,
                       pltpu.SemaphoreType.DMA,
                       pltpu.SemaphoreType.DMA],
        compiler_params=pltpu.CompilerParams(needs_layout_passes=False),
    )(_dispatch_body)
    return f(xf, pk_flat, poffi)


NSPLIT = 4
D_SP = D // NSPLIT
H_SP = H // NSPLIT


def _issue_load(e, b, we1, we2, w1buf, w2buf, sem1, sem2):
    @pl.when(e < E)
    def _():
        for s in range(NSPLIT):
            pltpu.make_async_copy(we1.at[e, pl.ds(s * D_SP, D_SP)],
                                  w1buf.at[b, pl.ds(s * D_SP, D_SP)],
                                  sem1.at[b, s]).start()
            pltpu.make_async_copy(we2.at[e, pl.ds(s * H_SP, H_SP)],
                                  w2buf.at[b, pl.ds(s * H_SP, H_SP)],
                                  sem2.at[b, s]).start()


def _wait_load(b, we1, we2, w1buf, w2buf, sem1, sem2):
    for s in range(NSPLIT):
        pltpu.make_async_copy(we1.at[0, pl.ds(s * D_SP, D_SP)],
                              w1buf.at[b, pl.ds(s * D_SP, D_SP)],
                              sem1.at[b, s]).wait()
        pltpu.make_async_copy(we2.at[0, pl.ds(s * H_SP, H_SP)],
                              w2buf.at[b, pl.ds(s * H_SP, H_SP)],
                              sem2.at[b, s]).wait()


def _gmm_body(te_s, first_s, par_s, nxt_s, xs_ref, we1, we2,
              ys_ref, w1buf, w2buf, sem1, sem2):
    t = pl.program_id(0)

    @pl.when(t == 0)
    def _():
        _issue_load(te_s[0], 0, we1, we2, w1buf, w2buf, sem1, sem2)

    @pl.when(first_s[t] == 1)
    def _():
        b = par_s[t]
        _wait_load(b, we1, we2, w1buf, w2buf, sem1, sem2)
        _issue_load(nxt_s[t], 1 - b, we1, we2, w1buf, w2buf, sem1, sem2)

    b = par_s[t]
    h = jax.nn.silu(jnp.dot(xs_ref[...], w1buf[b],
                            preferred_element_type=jnp.float32))
    ys_ref[...] = jnp.dot(h, w2buf[b], preferred_element_type=jnp.float32)


def _gmm(te, first, par, nxt, xs, We1, We2):
    grid_spec = pltpu.PrefetchScalarGridSpec(
        num_scalar_prefetch=4,
        grid=(NTILES_MOE,),
        in_specs=[pl.BlockSpec((TM, D), lambda t, *_: (t, 0)),
                  pl.BlockSpec(memory_space=pl.ANY),
                  pl.BlockSpec(memory_space=pl.ANY)],
        out_specs=pl.BlockSpec((TM, D), lambda t, *_: (t, 0)),
        scratch_shapes=[pltpu.VMEM((2, D, H), jnp.float32),
                        pltpu.VMEM((2, H, D), jnp.float32),
                        pltpu.SemaphoreType.DMA((2, NSPLIT)),
                        pltpu.SemaphoreType.DMA((2, NSPLIT))],
    )
    return pl.pallas_call(
        _gmm_body,
        grid_spec=grid_spec,
        out_shape=jax.ShapeDtypeStruct((CAP, D), jnp.float32),
        compiler_params=pltpu.CompilerParams(
            vmem_limit_bytes=100 * 1024 * 1024),
    )(te, first, par, nxt, xs, We1, We2)


def _shared_body(x_ref, w1_ref, w2_ref, o_ref):
    h = jax.nn.silu(jnp.dot(x_ref[...], w1_ref[...],
                            preferred_element_type=jnp.float32))
    o_ref[...] = jnp.dot(h, w2_ref[...], preferred_element_type=jnp.float32)


def _shared(xf, Ws1, Ws2):
    return pl.pallas_call(
        _shared_body,
        grid=(NTOK_TILES,),
        in_specs=[pl.BlockSpec((TM, D), lambda i: (i, 0)),
                  pl.BlockSpec((D, H), lambda i: (0, 0)),
                  pl.BlockSpec((H, D), lambda i: (0, 0))],
        out_specs=pl.BlockSpec((TM, D), lambda i: (i, 0)),
        out_shape=jax.ShapeDtypeStruct((N, D), jnp.float32),
    )(xf, Ws1, Ws2)


def _combine_body(ys_hbm, sh_hbm, pos0_hbm, pos1_hbm, pk_hbm,
                  out_hbm,
                  idx0_v, idx1_v, pk_v, y0_v, y1_v, s_v, o_v,
                  semL, semW0, semW1):
    semW = (semW0, semW1)
    wid = lax.axis_index("s") * NC + lax.axis_index("c")
    base = wid * TPW
    d1 = pltpu.async_copy(pos0_hbm.at[pl.ds(base, TPW)], idx0_v, semL)
    d2 = pltpu.async_copy(pos1_hbm.at[pl.ds(base, TPW)], idx1_v, semL)
    d3 = pltpu.async_copy(pk_hbm.at[pl.ds(base * E, TPW * E)], pk_v, semL)
    d1.wait(); d2.wait(); d3.wait()

    def issue(c):
        p = c % 2
        sl = pl.ds(c * CHUNK, CHUNK)
        return (
            pltpu.async_copy(ys_hbm.at[idx0_v.at[sl]], y0_v.at[p], semL),
            pltpu.async_copy(ys_hbm.at[idx1_v.at[sl]], y1_v.at[p], semL),
            pltpu.async_copy(sh_hbm.at[pl.ds(base + c * CHUNK, CHUNK)],
                             s_v.at[p], semL),
        )

    descs = issue(0)
    wdescs = [None, None]
    for c in range(NCHUNK):
        p = c % 2
        for dd in descs:
            dd.wait()
        if c + 1 < NCHUNK:
            descs = issue(c + 1)
        if wdescs[p] is not None:
            wdescs[p].wait()

        def token_body(j, _):
            jj = jnp.full((16,), (c * CHUNK + j) * E, jnp.int32)
            gj0 = plsc.load_gather(pk_v, [jj + 2])
            gj1 = plsc.load_gather(pk_v, [jj + 3])
            for k in range(D // 16):
                sl = pl.ds(16 * k, 16)
                o_v[p, j, sl] = (s_v[p, j, sl] + gj0 * y0_v[p, j, sl]
                                 + gj1 * y1_v[p, j, sl])
            return 0

        lax.fori_loop(0, CHUNK, token_body, 0)
        wdescs[p] = pltpu.async_copy(
            o_v.at[p], out_hbm.at[pl.ds(base + c * CHUNK, CHUNK)], semW[p])
    for wd in wdescs:
        if wd is not None:
            wd.wait()


def _combine(ys, sh, pos0, pos1, pk_flat):
    mesh = plsc.VectorSubcoreMesh(core_axis_name="c", subcore_axis_name="s")
    f = functools.partial(
        pl.kernel,
        out_type=jax.ShapeDtypeStruct((N, D), jnp.float32),
        mesh=mesh,
        scratch_types=[pltpu.VMEM((TPW,), jnp.int32),
                       pltpu.VMEM((TPW,), jnp.int32),
                       pltpu.VMEM((TPW * E,), jnp.float32),
                       pltpu.VMEM((2, CHUNK, D), jnp.float32),
                       pltpu.VMEM((2, CHUNK, D), jnp.float32),
                       pltpu.VMEM((2, CHUNK, D), jnp.float32),
                       pltpu.VMEM((2, CHUNK, D), jnp.float32),
                       pltpu.SemaphoreType.DMA,
                       pltpu.SemaphoreType.DMA,
                       pltpu.SemaphoreType.DMA],
        compiler_params=pltpu.CompilerParams(needs_layout_passes=False),
    )(_combine_body)
    return f(ys, sh, pos0, pos1, pk_flat)


def kernel(x, Ws1, Ws2, We1, We2, Wg):
    Bn, Sn, Dn = x.shape
    xf = x.reshape(N, D)
    pk, poff, te, first, par, nxt = _routing(xf, Wg)
    pk_flat = pk.reshape(N * E)

    xs, pos0, pos1 = _dispatch(xf, pk_flat, poff.reshape(64))
    sh = _shared(xf, Ws1, Ws2)
    ys = _gmm(te.reshape(64), first.reshape(64), par.reshape(64),
              nxt.reshape(64), xs, We1, We2)
    out = _combine(ys, sh, pos0, pos1, pk_flat)
    return out.reshape(Bn, Sn, Dn)
